# Initial kernel scaffold; baseline (speedup 1.0000x reference)
import jax
import jax.numpy as jnp
from jax.experimental import pallas as pl


def kernel(center_past1, center_past2, time_diff):
    raise NotImplementedError("write your pallas kernel here")


# speedup vs baseline: 18.9285x; 18.9285x over previous
import functools

import jax
import jax.numpy as jnp
from jax import lax
from jax.experimental import pallas as pl
from jax.experimental.pallas import tpu as pltpu
from jax.experimental.pallas import tpu_sc as plsc

N = 512
THRE = 20.0
INF = 1e18
BIG = 1 << 30
NCHUNK = N // 16


def _cost_body(cp1t_ref, cp2_ref, cp2t_ref, cp1_ref, cost_ref, costt_ref,
               m2_ref, m1_ref):
    x1r = cp1t_ref[0:1, :]
    y1r = cp1t_ref[1:2, :]
    x2c = cp2_ref[:, 0:1]
    y2c = cp2_ref[:, 1:2]
    dx = x2c - x1r
    dy = y2c - y1r
    cost = jnp.sqrt(dx * dx + dy * dy)
    cost_ref[...] = cost
    x2r = cp2t_ref[0:1, :]
    y2r = cp2t_ref[1:2, :]
    x1c = cp1_ref[:, 0:1]
    y1c = cp1_ref[:, 1:2]
    dxt = x1c - x2r
    dyt = y1c - y2r
    costt = jnp.sqrt(dxt * dxt + dyt * dyt)
    costt_ref[...] = costt
    m2_ref[...] = (jnp.min(costt, axis=0, keepdims=True) <= THRE).astype(jnp.float32)
    m1_ref[...] = (jnp.min(cost, axis=0, keepdims=True) <= THRE).astype(jnp.float32)


def _cost_stage(cp1, cp2):
    return pl.pallas_call(
        _cost_body,
        out_shape=(
            jax.ShapeDtypeStruct((N, N), jnp.float32),
            jax.ShapeDtypeStruct((N, N), jnp.float32),
            jax.ShapeDtypeStruct((1, N), jnp.float32),
            jax.ShapeDtypeStruct((1, N), jnp.float32),
        ),
    )(cp1.T, cp2, cp2.T, cp1)


def _sload(ref, idx):
    return plsc.load_gather(ref, [jnp.full((16,), idx, jnp.int32)])[0]


def _sc_matcher(cost, costt, mask2, mask1, x1, y1, x2, y2, scal):
    mesh = plsc.VectorSubcoreMesh(core_axis_name="c", subcore_axis_name="s",
                                  num_cores=2, num_subcores=16)

    @functools.partial(
        pl.kernel,
        out_type=(
            jax.ShapeDtypeStruct((N,), jnp.float32),
            jax.ShapeDtypeStruct((N,), jnp.float32),
        ),
        mesh=mesh,
        compiler_params=pltpu.CompilerParams(needs_layout_passes=False),
        scratch_types=dict(
            c_sh=pltpu.VMEM_SHARED((2, N, N), jnp.float32),
            row_buf=pltpu.VMEM((N,), jnp.float32),
            vcol=pltpu.VMEM((N,), jnp.float32),
            minv=pltpu.VMEM((N,), jnp.float32),
            wayb=pltpu.VMEM((N,), jnp.int32),
            freef=pltpu.VMEM((N,), jnp.float32),
            basef=pltpu.VMEM((N,), jnp.float32),
            pcol=pltpu.VMEM((N,), jnp.int32),
            urow=pltpu.VMEM((N,), jnp.float32),
            colg=pltpu.VMEM((N,), jnp.int32),
            ridx=pltpu.VMEM((N,), jnp.int32),
            idx1b=pltpu.VMEM((N + 16,), jnp.int32),
            idx2b=pltpu.VMEM((N + 16,), jnp.int32),
            cola=pltpu.VMEM((N,), jnp.int32),
            sbuf=pltpu.VMEM((N,), jnp.int32),
            orderb=pltpu.VMEM((N,), jnp.int32),
            m1v=pltpu.VMEM((N,), jnp.float32),
            m2v=pltpu.VMEM((N,), jnp.float32),
            x1v=pltpu.VMEM((N,), jnp.float32),
            y1v=pltpu.VMEM((N,), jnp.float32),
            x2v=pltpu.VMEM((N,), jnp.float32),
            y2v=pltpu.VMEM((N,), jnp.float32),
            outxv=pltpu.VMEM((N,), jnp.float32),
            outyv=pltpu.VMEM((N,), jnp.float32),
            scalv=pltpu.VMEM((16,), jnp.float32),
        ),
    )
    def k(cost_hbm, costt_hbm, m2_hbm, m1_hbm, x1_hbm, y1_hbm, x2_hbm,
          y2_hbm, scal_hbm, outx_hbm, outy_hbm, c_sh, row_buf, vcol, minv,
          wayb, freef, basef, pcol, urow, colg, ridx, idx1b, idx2b, cola,
          sbuf, orderb, m1v, m2v, x1v, y1v, x2v, y2v, outxv, outyv, scalv):
        cid = lax.axis_index("c")
        sid = lax.axis_index("s")
        lane = lax.iota(jnp.int32, 16)

        @pl.when(jnp.logical_and(cid == 0, sid == 0))
        def _():
            pltpu.sync_copy(cost_hbm, c_sh.at[0])
            pltpu.sync_copy(costt_hbm, c_sh.at[1])
            pltpu.sync_copy(m2_hbm, m2v)
            pltpu.sync_copy(m1_hbm, m1v)
            pltpu.sync_copy(x1_hbm, x1v)
            pltpu.sync_copy(y1_hbm, y1v)
            pltpu.sync_copy(x2_hbm, x2v)
            pltpu.sync_copy(y2_hbm, y2v)
            pltpu.sync_copy(scal_hbm, scalv)

            def comp_body(c, offs):
                o1, o2 = offs
                ds = pl.ds(c * 16, 16)
                gi = lane + c * 16
                mk1 = m1v[ds] > 0.5
                mk2 = m2v[ds] > 0.5
                plsc.store_compressed(idx1b.at[pl.ds(o1, 16)], gi, mask=mk1)
                plsc.store_compressed(idx2b.at[pl.ds(o2, 16)], gi, mask=mk2)
                o1 = o1 + jnp.sum(mk1.astype(jnp.int32))
                o2 = o2 + jnp.sum(mk2.astype(jnp.int32))
                return o1, o2

            def initpad_body(c, _):
                ds = pl.ds(c * 16, 16)
                idx1b[ds] = jnp.full((16,), N, jnp.int32)
                idx2b[ds] = jnp.full((16,), N, jnp.int32)
                return 0

            lax.fori_loop(0, NCHUNK + 1, initpad_body, 0)
            n1, n2 = lax.fori_loop(0, NCHUNK, comp_body,
                                   (jnp.int32(0), jnp.int32(0)))
            t = n2 > n1
            nn = jnp.where(t, n1, n2)
            mm = jnp.where(t, n2, n1)
            tsel = t.astype(jnp.int32)

            def gid_body(c, _):
                ds = pl.ds(c * 16, 16)
                g1 = jnp.clip(idx1b[ds], 0, N - 1)
                g2 = jnp.clip(idx2b[ds], 0, N - 1)
                ridx[ds] = jnp.where(t, g1, g2)
                colg[ds] = jnp.where(t, g2, g1)
                pcol[ds] = jnp.zeros((16,), jnp.int32)
                urow[ds] = jnp.zeros((16,), jnp.float32)
                vcol[ds] = jnp.zeros((16,), jnp.float32)
                wayb[ds] = jnp.zeros((16,), jnp.int32)
                cola[ds] = jnp.zeros((16,), jnp.int32)
                sbuf[ds] = jnp.full((16,), BIG, jnp.int32)
                return 0

            lax.fori_loop(0, NCHUNK, gid_body, 0)

            def row_body(i, _):
                @pl.when(i <= nn)
                def _():
                    def reset_body(c, _):
                        ds = pl.ds(c * 16, 16)
                        minv[ds] = jnp.full((16,), INF, jnp.float32)
                        freef[ds] = jnp.where(lane + c * 16 < mm,
                                              jnp.float32(1.0), jnp.float32(0.0))
                        return 0

                    lax.fori_loop(0, NCHUNK, reset_body, 0)

                    def wcond(carry):
                        return jnp.logical_not(carry[3])

                    def wbody(carry):
                        j0, pending, dsum, _ = carry
                        lane0 = lane < 1

                        @pl.when(j0 > 0)
                        def _():
                            j0v = jnp.full((16,), j0 - 1, jnp.int32)
                            plsc.store_scatter(freef, [j0v],
                                               jnp.zeros((16,), jnp.float32),
                                               mask=lane0)
                            plsc.store_scatter(basef, [j0v],
                                               jnp.full((16,), dsum,
                                                        jnp.float32),
                                               mask=lane0)

                        i0 = jnp.where(j0 > 0,
                                       _sload(pcol, jnp.maximum(j0 - 1, 0)), i)
                        u_i0 = _sload(urow, i0 - 1)
                        rowid = _sload(ridx, i0 - 1)
                        pltpu.sync_copy(c_sh.at[tsel, rowid], row_buf)

                        def chunk_body(c, bc):
                            bestv, besti = bc
                            ds = pl.ds(c * 16, 16)
                            gidx = lane + c * 16
                            crow = plsc.load_gather(row_buf, [colg[ds]])
                            cur = crow - u_i0 - vcol[ds]
                            frm = freef[ds] > 0.5
                            mv = minv[ds]
                            m_eff = jnp.where(frm, mv - pending, mv)
                            upd = jnp.logical_and(frm, cur < m_eff)
                            nmv = jnp.where(upd, cur, m_eff)
                            minv[ds] = nmv
                            wayb[ds] = jnp.where(upd, j0, wayb[ds])
                            cand = jnp.where(frm, nmv, INF)
                            ltm = cand < bestv
                            bestv = jnp.where(ltm, cand, bestv)
                            besti = jnp.where(ltm, gidx, besti)
                            return bestv, besti

                        bestv0 = jnp.full((16,), INF, jnp.float32)
                        besti0 = jnp.full((16,), BIG, jnp.int32)
                        bestv, besti = lax.fori_loop(0, NCHUNK, chunk_body,
                                                     (bestv0, besti0))
                        mval = jnp.min(bestv)
                        candidx = jnp.where(bestv == mval, besti, BIG)
                        jarg = jnp.min(candidx)
                        j1 = jarg + 1
                        delta = mval
                        done = _sload(pcol, jarg) == 0
                        return j1, delta, dsum + delta, done

                    j0f, _, dsum_f, _ = lax.while_loop(
                        wcond, wbody,
                        (jnp.int32(0), jnp.float32(0.0), jnp.float32(0.0),
                         jnp.bool_(False)))

                    plsc.addupdate_scatter(urow,
                                           [jnp.full((16,), i - 1, jnp.int32)],
                                           jnp.full((16,), dsum_f, jnp.float32),
                                           mask=lane < 1)

                    def upd_body(c, _):
                        ds = pl.ds(c * 16, 16)
                        gidx = lane + c * 16
                        usedm = jnp.logical_and(freef[ds] < 0.5, gidx < mm)
                        amt = jnp.where(usedm, dsum_f - basef[ds],
                                        jnp.float32(0.0))
                        vcol[ds] = vcol[ds] - amt
                        pc = jnp.maximum(pcol[ds] - 1, 0)
                        plsc.addupdate_scatter(urow, [pc], amt, mask=usedm)
                        return 0

                    lax.fori_loop(0, NCHUNK, upd_body, 0)

                    def acond(carry):
                        return carry[0] != 0

                    def abody(carry):
                        j0, _ = carry
                        jw = _sload(wayb, j0 - 1)
                        pv = jnp.where(jw > 0,
                                       _sload(pcol, jnp.maximum(jw - 1, 0)), i)
                        plsc.store_scatter(pcol,
                                           [jnp.full((16,), j0 - 1, jnp.int32)],
                                           jnp.full((16,), pv, jnp.int32),
                                           mask=lane < 1)
                        return jw, 0

                    lax.while_loop(acond, abody, (j0f, 0))
                return 0

            lax.fori_loop(1, N + 1, row_body, 0)

            def cola_body(c, _):
                ds = pl.ds(c * 16, 16)
                gidx = lane + c * 16
                pc = pcol[ds]
                ok = jnp.logical_and(pc > 0, gidx < mm)
                plsc.store_scatter(cola, [jnp.maximum(pc - 1, 0)], gidx,
                                   mask=ok)
                return 0

            lax.fori_loop(0, NCHUNK, cola_body, 0)

            def sscat_body(c, _):
                ds = pl.ds(c * 16, 16)
                gidx = lane + c * 16
                val = gidx < nn
                cx = jnp.where(val, cola[ds], N)
                plsc.store_scatter(sbuf, [jnp.minimum(cx, N - 1)], gidx,
                                   mask=val)
                return 0

            lax.fori_loop(0, NCHUNK, sscat_body, 0)

            def order_body(c, run):
                ds = pl.ds(c * 16, 16)
                gidx = lane + c * 16
                orderb[ds] = gidx
                return run

            lax.fori_loop(0, NCHUNK, order_body, 0)

            def rank_body(c, run):
                ds = pl.ds(c * 16, 16)
                sv = sbuf[ds]
                occ = sv != BIG
                occ_i = occ.astype(jnp.int32)
                cs = plsc.cumsum(occ_i)
                rank = run + cs - occ_i
                plsc.store_scatter(orderb, [jnp.minimum(rank, N - 1)], sv,
                                   mask=occ)
                return run + jnp.sum(occ_i)

            lax.fori_loop(0, NCHUNK, rank_body, jnp.int32(0))

            sv = scalv[...]
            tl = sv[0]
            td0 = sv[1]

            def fin_init(c, _):
                ds = pl.ds(c * 16, 16)
                outxv[ds] = x1v[ds]
                outyv[ds] = y1v[ds]
                return 0

            lax.fori_loop(0, NCHUNK, fin_init, 0)

            def fin_body(c, _):
                ds = pl.ds(c * 16, 16)
                gidx = lane + c * 16
                valid = gidx < nn
                cx = jnp.where(valid, cola[ds], N)
                odr = orderb[ds]
                cxo = plsc.load_gather(cola, [jnp.minimum(odr, N - 1)])
                cxo = jnp.where(odr < nn, cxo, N)
                r = jnp.where(t, cxo, gidx)
                cc = jnp.where(t, odr, cx)
                p2 = plsc.load_gather(idx2b, [jnp.clip(r, 0, N - 1)])
                p1 = plsc.load_gather(idx1b, [jnp.clip(cc, 0, N - 1)])
                p2c = jnp.clip(p2, 0, N - 1)
                p1c = jnp.clip(p1, 0, N - 1)
                mx2 = plsc.load_gather(x2v, [p2c])
                my2 = plsc.load_gather(y2v, [p2c])
                mx1 = plsc.load_gather(x1v, [p1c])
                my1 = plsc.load_gather(y1v, [p1c])
                ex = mx1 + (mx1 - mx2) / tl * (jnp.float32(0.0) - td0)
                ey = my1 + (my1 - my2) / tl * (jnp.float32(0.0) - td0)
                okm = jnp.logical_and(valid, p1 < N)
                plsc.store_scatter(outxv, [p1c], ex, mask=okm)
                plsc.store_scatter(outyv, [p1c], ey, mask=okm)
                return 0

            lax.fori_loop(0, NCHUNK, fin_body, 0)

            pltpu.sync_copy(outxv, outx_hbm)
            pltpu.sync_copy(outyv, outy_hbm)

    return k(cost, costt, mask2, mask1, x1, y1, x2, y2, scal)


_TEMPLATE = [(1, -1, -1), (1, 1, -1), (-1, 1, -1), (-1, -1, -1),
             (1, -1, 1), (1, 1, 1), (-1, 1, 1), (-1, -1, 1)]


def _corners_body(cpt_ref, ox_ref, oy_ref, out_ref):
    x = ox_ref[0:1, :]
    y = oy_ref[0:1, :]
    z = cpt_ref[2:3, :]
    h = cpt_ref[3:4, :]
    w = cpt_ref[4:5, :]
    ll = cpt_ref[5:6, :]
    yaw = cpt_ref[6:7, :]
    cy = jnp.cos(yaw)
    sy = jnp.sin(yaw)
    for kk, (tx, ty, tz) in enumerate(_TEMPLATE):
        cxk = ll * (0.5 * tx)
        cyk = w * (0.5 * ty)
        czk = h * (0.5 * tz)
        rx = cxk * cy - cyk * sy + x
        ry = cxk * sy + cyk * cy + y
        rz = czk + z
        out_ref[3 * kk + 0:3 * kk + 1, :] = rx
        out_ref[3 * kk + 1:3 * kk + 2, :] = ry
        out_ref[3 * kk + 2:3 * kk + 3, :] = rz


def _corners_stage(center_t, ox, oy):
    return pl.pallas_call(
        _corners_body,
        out_shape=jax.ShapeDtypeStruct((24, N), jnp.float32),
    )(center_t, ox.reshape(1, N), oy.reshape(1, N))


def kernel(center_past1, center_past2, time_diff):
    cp1 = center_past1[:, :2]
    cp2 = center_past2[:, :2]
    cost, costt, m2, m1 = _cost_stage(cp1, cp2)
    tl = time_diff[0] - time_diff[1]
    tl = jnp.where(tl == 0, jnp.float32(1.0), tl)
    scal = jnp.zeros((16,), jnp.float32).at[0].set(tl).at[1].set(time_diff[0])
    outx, outy = _sc_matcher(
        cost, costt, m2.reshape(N), m1.reshape(N),
        cp1[:, 0], cp1[:, 1], cp2[:, 0], cp2[:, 1], scal)
    center_out = jnp.concatenate(
        [outx[:, None], outy[:, None], center_past1[:, 2:]], axis=1)
    out24 = _corners_stage(center_out.T, outx, outy)
    corners = out24.reshape(8, 3, N).transpose(2, 0, 1)
    return center_out, corners


# speedup vs baseline: 31.3124x; 1.6543x over previous
import functools

import jax
import jax.numpy as jnp
from jax import lax
from jax.experimental import pallas as pl
from jax.experimental.pallas import tpu as pltpu
from jax.experimental.pallas import tpu_sc as plsc

N = 512
THRE = 20.0
INF = 1e18
BIG = 1 << 30
NCHUNK = N // 16
NT = 16
CPT = N // NT
_STAGE = 4


def _cost_body(cp1t_ref, cp2_ref, cp2t_ref, cp1_ref, cost_ref, costt_ref,
               m2_ref, m1_ref):
    x1r = cp1t_ref[0:1, :]
    y1r = cp1t_ref[1:2, :]
    x2c = cp2_ref[:, 0:1]
    y2c = cp2_ref[:, 1:2]
    dx = x2c - x1r
    dy = y2c - y1r
    cost = jnp.sqrt(dx * dx + dy * dy)
    cost_ref[...] = cost
    x2r = cp2t_ref[0:1, :]
    y2r = cp2t_ref[1:2, :]
    x1c = cp1_ref[:, 0:1]
    y1c = cp1_ref[:, 1:2]
    dxt = x1c - x2r
    dyt = y1c - y2r
    costt = jnp.sqrt(dxt * dxt + dyt * dyt)
    costt_ref[...] = costt
    m2_ref[...] = (jnp.min(costt, axis=0, keepdims=True) <= THRE).astype(jnp.float32)
    m1_ref[...] = (jnp.min(cost, axis=0, keepdims=True) <= THRE).astype(jnp.float32)


def _cost_stage(cp1, cp2):
    return pl.pallas_call(
        _cost_body,
        out_shape=(
            jax.ShapeDtypeStruct((N, N), jnp.float32),
            jax.ShapeDtypeStruct((N, N), jnp.float32),
            jax.ShapeDtypeStruct((1, N), jnp.float32),
            jax.ShapeDtypeStruct((1, N), jnp.float32),
        ),
    )(cp1.T, cp2, cp2.T, cp1)


def _sload(ref, idx):
    return plsc.load_gather(ref, [jnp.full((16,), idx, jnp.int32)])[0]


def _sload2(ref, r, c):
    return plsc.load_gather(ref, [jnp.full((16,), r, jnp.int32),
                                  jnp.full((16,), c, jnp.int32)])[0]


def _sc_matcher(cost, costt, mask2, mask1, x1, y1, x2, y2, scal):
    mesh = plsc.VectorSubcoreMesh(core_axis_name="c", subcore_axis_name="s",
                                  num_cores=2, num_subcores=16)

    @functools.partial(
        pl.kernel,
        out_type=(
            jax.ShapeDtypeStruct((N,), jnp.float32),
            jax.ShapeDtypeStruct((N,), jnp.float32),
        ),
        mesh=mesh,
        compiler_params=pltpu.CompilerParams(needs_layout_passes=False),
        scratch_types=dict(
            p_sh=pltpu.VMEM_SHARED((NT, 128), jnp.float32),
            wam_sh=pltpu.VMEM_SHARED((NT, 128), jnp.int32),
            cw=pltpu.VMEM((N, CPT), jnp.float32),
            rbuf=pltpu.VMEM((64, N), jnp.float32),
            minvL=pltpu.VMEM((CPT,), jnp.float32),
            basefL=pltpu.VMEM((CPT,), jnp.float32),
            wayL=pltpu.VMEM((CPT,), jnp.int32),
            vcolL=pltpu.VMEM((CPT,), jnp.float32),
            amtL=pltpu.VMEM((CPT,), jnp.float32),
            wamF=pltpu.VMEM((NT, 128), jnp.int32),
            wam128=pltpu.VMEM((128,), jnp.int32),
            pcol=pltpu.VMEM((N,), jnp.int32),
            urow=pltpu.VMEM((N,), jnp.float32),
            colg=pltpu.VMEM((N,), jnp.int32),
            ridx=pltpu.VMEM((N,), jnp.int32),
            idx1b=pltpu.VMEM((N + 16,), jnp.int32),
            idx2b=pltpu.VMEM((N + 16,), jnp.int32),
            cola=pltpu.VMEM((N,), jnp.int32),
            sbuf=pltpu.VMEM((N,), jnp.int32),
            orderb=pltpu.VMEM((N,), jnp.int32),
            m1v=pltpu.VMEM((N,), jnp.float32),
            m2v=pltpu.VMEM((N,), jnp.float32),
            x1v=pltpu.VMEM((N,), jnp.float32),
            y1v=pltpu.VMEM((N,), jnp.float32),
            x2v=pltpu.VMEM((N,), jnp.float32),
            y2v=pltpu.VMEM((N,), jnp.float32),
            outxv=pltpu.VMEM((N,), jnp.float32),
            outyv=pltpu.VMEM((N,), jnp.float32),
            scalv=pltpu.VMEM((16,), jnp.float32),
            pbuf=pltpu.VMEM((128,), jnp.float32),
            xbuf=pltpu.VMEM((NT, 128), jnp.float32),
            sem=pltpu.SemaphoreType.DMA,
        ),
    )
    def k(cost_hbm, costt_hbm, m2_hbm, m1_hbm, x1_hbm, y1_hbm, x2_hbm,
          y2_hbm, scal_hbm, outx_hbm, outy_hbm, p_sh, wam_sh, cw, rbuf,
          minvL, basefL, wayL, vcolL, amtL, wamF, wam128, pcol, urow, colg,
          ridx, idx1b, idx2b, cola, sbuf, orderb, m1v, m2v, x1v, y1v, x2v,
          y2v, outxv, outyv, scalv, pbuf, xbuf, sem):
        cid = lax.axis_index("c")
        sid = lax.axis_index("s")
        lane = lax.iota(jnp.int32, 16)

        @pl.when(cid == 0)
        def _():
            pltpu.sync_copy(m2_hbm, m2v)
            pltpu.sync_copy(m1_hbm, m1v)

            def initpad_body(c, _):
                ds = pl.ds(c * 16, 16)
                idx1b[ds] = jnp.full((16,), N, jnp.int32)
                idx2b[ds] = jnp.full((16,), N, jnp.int32)
                return 0

            lax.fori_loop(0, NCHUNK + 1, initpad_body, 0)

            def comp_body(c, offs):
                o1, o2 = offs
                ds = pl.ds(c * 16, 16)
                gi = lane + c * 16
                mk1 = m1v[ds] > 0.5
                mk2 = m2v[ds] > 0.5
                plsc.store_compressed(idx1b.at[pl.ds(o1, 16)], gi, mask=mk1)
                plsc.store_compressed(idx2b.at[pl.ds(o2, 16)], gi, mask=mk2)
                o1 = o1 + jnp.sum(mk1.astype(jnp.int32))
                o2 = o2 + jnp.sum(mk2.astype(jnp.int32))
                return o1, o2

            n1, n2 = lax.fori_loop(0, NCHUNK, comp_body,
                                   (jnp.int32(0), jnp.int32(0)))
            t = n2 > n1
            nn = jnp.where(t, n1, n2)
            mm = jnp.where(t, n2, n1)

            def gid_body(c, _):
                ds = pl.ds(c * 16, 16)
                g1 = jnp.clip(idx1b[ds], 0, N - 1)
                g2 = jnp.clip(idx2b[ds], 0, N - 1)
                ridx[ds] = jnp.where(t, g1, g2)
                colg[ds] = jnp.where(t, g2, g1)
                pcol[ds] = jnp.zeros((16,), jnp.int32)
                urow[ds] = jnp.zeros((16,), jnp.float32)
                cola[ds] = jnp.zeros((16,), jnp.int32)
                sbuf[ds] = jnp.full((16,), BIG, jnp.int32)
                return 0

            lax.fori_loop(0, NCHUNK, gid_body, 0)

            colgA = colg[pl.ds(sid * CPT, 16)]
            colgB = colg[pl.ds(sid * CPT + 16, 16)]
            vcolL[pl.ds(0, 16)] = jnp.zeros((16,), jnp.float32)
            vcolL[pl.ds(16, 16)] = jnp.zeros((16,), jnp.float32)
            wayL[pl.ds(0, 16)] = jnp.zeros((16,), jnp.int32)
            wayL[pl.ds(16, 16)] = jnp.zeros((16,), jnp.int32)

            def wfinit_body(c, _):
                wamF[c, pl.ds(0, 16)] = jnp.zeros((16,), jnp.int32)
                wamF[c, pl.ds(16, 16)] = jnp.zeros((16,), jnp.int32)
                wamF[c, pl.ds(32, 16)] = jnp.zeros((16,), jnp.int32)
                wamF[c, pl.ds(48, 16)] = jnp.zeros((16,), jnp.int32)
                return 0

            lax.fori_loop(0, NT, wfinit_body, 0)
            for b in range(8):
                idx_slice = ridx.at[pl.ds(b * 64, 64)]

                @pl.when(t)
                def _():
                    pltpu.async_copy(costt_hbm.at[idx_slice], rbuf, sem).wait()

                @pl.when(jnp.logical_not(t))
                def _():
                    pltpu.async_copy(cost_hbm.at[idx_slice], rbuf, sem).wait()

                def stg_body(r8, _):
                    r8v = jnp.full((16,), r8, jnp.int32)
                    va = plsc.load_gather(rbuf, [r8v, colgA])
                    vb = plsc.load_gather(rbuf, [r8v, colgB])
                    rr = b * 64 + r8
                    cw[rr, pl.ds(0, 16)] = va
                    cw[rr, pl.ds(16, 16)] = vb
                    return 0

                lax.fori_loop(0, 64, stg_body, 0)

            gcolA = lane + sid * CPT
            gcolB = gcolA + 16

            def row_body(i, parity0):
                def do_row(parity0):
                    minvL[pl.ds(0, 16)] = jnp.full((16,), INF, jnp.float32)
                    minvL[pl.ds(16, 16)] = jnp.full((16,), INF, jnp.float32)
                    basefL[pl.ds(0, 16)] = jnp.where(
                        gcolA < mm, jnp.float32(INF), jnp.float32(0.0))
                    basefL[pl.ds(16, 16)] = jnp.where(
                        gcolB < mm, jnp.float32(INF), jnp.float32(0.0))

                    def wcond(carry):
                        cap = 520 if _STAGE < 4 else 1 << 20
                        return jnp.logical_and(jnp.logical_not(carry[4]),
                                               carry[5] < cap)

                    def wbody(carry):
                        j0, pending, dsum, parity, _, cnt = carry
                        own = (j0 - 1) // CPT

                        @pl.when(jnp.logical_and(j0 > 0, own == sid))
                        def _():
                            kloc = jnp.full((16,), (j0 - 1) % CPT, jnp.int32)
                            plsc.store_scatter(
                                basefL, [kloc],
                                jnp.full((16,), dsum, jnp.float32),
                                mask=lane < 1)

                        i0 = jnp.where(j0 > 0,
                                       _sload(pcol, jnp.maximum(j0 - 1, 0)),
                                       i)
                        u_i0 = _sload(urow, i0 - 1)
                        rr = i0 - 1

                        bsfA = basefL[pl.ds(0, 16)]
                        frmA = bsfA >= jnp.float32(INF)
                        curA = cw[rr, pl.ds(0, 16)] - u_i0 - vcolL[pl.ds(0, 16)]
                        mvA = minvL[pl.ds(0, 16)]
                        meA = jnp.where(frmA, mvA - pending, mvA)
                        upA = jnp.logical_and(frmA, curA < meA)
                        nmA = jnp.where(upA, curA, meA)
                        minvL[pl.ds(0, 16)] = nmA
                        wayL[pl.ds(0, 16)] = jnp.where(upA, j0,
                                                       wayL[pl.ds(0, 16)])
                        candA = jnp.where(frmA, nmA, jnp.float32(INF))

                        bsfB = basefL[pl.ds(16, 16)]
                        frmB = bsfB >= jnp.float32(INF)
                        curB = cw[rr, pl.ds(16, 16)] - u_i0 - vcolL[pl.ds(16, 16)]
                        mvB = minvL[pl.ds(16, 16)]
                        meB = jnp.where(frmB, mvB - pending, mvB)
                        upB = jnp.logical_and(frmB, curB < meB)
                        nmB = jnp.where(upB, curB, meB)
                        minvL[pl.ds(16, 16)] = nmB
                        wayL[pl.ds(16, 16)] = jnp.where(upB, j0,
                                                        wayL[pl.ds(16, 16)])
                        candB = jnp.where(frmB, nmB, jnp.float32(INF))

                        ltm = candB < candA
                        bestv = jnp.where(ltm, candB, candA)
                        besti = jnp.where(ltm, gcolB, gcolA)
                        lmin = jnp.min(bestv)
                        lidx = jnp.min(jnp.where(bestv == lmin, besti, BIG))

                        if _STAGE >= 4:
                            pv = jnp.where(lane < 1,
                                           jnp.full((16,), lmin, jnp.float32),
                                           plsc.bitcast(jnp.full((16,), lidx,
                                                                 jnp.int32),
                                                        jnp.float32))
                            pbuf[pl.ds(0, 16)] = pv
                            pltpu.sync_copy(pbuf, p_sh.at[sid])
                            plsc.subcore_barrier()
                            pltpu.sync_copy(p_sh, xbuf)
                            mins = plsc.load_gather(
                                xbuf, [lane, jnp.zeros((16,), jnp.int32)])
                            idxs = plsc.bitcast(
                                plsc.load_gather(
                                    xbuf, [lane, jnp.ones((16,), jnp.int32)]),
                                jnp.int32)
                            delta = jnp.min(mins)
                            jarg = jnp.min(jnp.where(mins == delta, idxs, BIG))
                            plsc.subcore_barrier()
                        else:
                            if _STAGE == 3:
                                plsc.subcore_barrier()
                            delta = lmin
                            jarg = jnp.minimum(lidx, N - 1)
                        j1 = jarg + 1
                        done = _sload(pcol, jnp.minimum(jarg, N - 1)) == 0
                        if _STAGE == 3:
                            done = cnt >= 20
                        return (j1, delta, dsum + delta, 1 - parity, done,
                                cnt + 1)

                    j0f, _, dsum_f, parity_f, _, _ = lax.while_loop(
                        wcond, wbody,
                        (jnp.int32(0), jnp.float32(0.0), jnp.float32(0.0),
                         parity0, jnp.bool_(False), jnp.int32(0)))

                    plsc.addupdate_scatter(
                        urow, [jnp.full((16,), i - 1, jnp.int32)],
                        jnp.full((16,), dsum_f, jnp.float32),
                        mask=lane < 1)

                    usA = jnp.logical_and(basefL[pl.ds(0, 16)] < jnp.float32(INF),
                                          gcolA < mm)
                    aA = jnp.where(usA, dsum_f - basefL[pl.ds(0, 16)],
                                   jnp.float32(0.0))
                    vcolL[pl.ds(0, 16)] = vcolL[pl.ds(0, 16)] - aA
                    amtL[pl.ds(0, 16)] = aA
                    usB = jnp.logical_and(basefL[pl.ds(16, 16)] < jnp.float32(INF),
                                          gcolB < mm)
                    aB = jnp.where(usB, dsum_f - basefL[pl.ds(16, 16)],
                                   jnp.float32(0.0))
                    vcolL[pl.ds(16, 16)] = vcolL[pl.ds(16, 16)] - aB
                    amtL[pl.ds(16, 16)] = aB

                    wam128[pl.ds(0, 16)] = wayL[pl.ds(0, 16)]
                    wam128[pl.ds(16, 16)] = wayL[pl.ds(16, 16)]
                    wam128[pl.ds(32, 16)] = plsc.bitcast(amtL[pl.ds(0, 16)],
                                                         jnp.int32)
                    wam128[pl.ds(48, 16)] = plsc.bitcast(amtL[pl.ds(16, 16)],
                                                         jnp.int32)
                    if _STAGE >= 4:
                        pltpu.sync_copy(wam128, wam_sh.at[sid])
                        plsc.subcore_barrier()
                        pltpu.sync_copy(wam_sh, wamF)
                    else:
                        wamF[sid, pl.ds(0, 16)] = wam128[pl.ds(0, 16)]
                        wamF[sid, pl.ds(16, 16)] = wam128[pl.ds(16, 16)]
                        wamF[sid, pl.ds(32, 16)] = wam128[pl.ds(32, 16)]
                        wamF[sid, pl.ds(48, 16)] = wam128[pl.ds(48, 16)]

                    def uupd_body(c, _):
                        a16A = plsc.bitcast(wamF[c, pl.ds(32, 16)],
                                            jnp.float32)
                        a16B = plsc.bitcast(wamF[c, pl.ds(48, 16)],
                                            jnp.float32)
                        pcA = jnp.maximum(pcol[pl.ds(c * CPT, 16)] - 1, 0)
                        pcB = jnp.maximum(pcol[pl.ds(c * CPT + 16, 16)] - 1, 0)
                        plsc.addupdate_scatter(urow, [pcA], a16A,
                                               mask=a16A != 0)
                        plsc.addupdate_scatter(urow, [pcB], a16B,
                                               mask=a16B != 0)
                        return 0

                    lax.fori_loop(0, NT, uupd_body, 0)

                    def acond(carry):
                        return jnp.logical_and(carry[0] != 0,
                                               carry[1] < N + 2)

                    def abody(carry):
                        j0, hop = carry
                        jc = jnp.clip(j0 - 1, 0, N - 1)
                        jw = _sload2(wamF, jc // CPT, jc % CPT)
                        pv2 = jnp.where(
                            jw > 0, _sload(pcol, jnp.maximum(jw - 1, 0)), i)
                        plsc.store_scatter(
                            pcol, [jnp.full((16,), jc, jnp.int32)],
                            jnp.full((16,), pv2, jnp.int32),
                            mask=lane < 1)
                        return jw, hop + 1

                    lax.while_loop(acond, abody, (j0f, jnp.int32(0)))
                    if _STAGE >= 4:
                        plsc.subcore_barrier()
                    return parity_f

                if _STAGE < 2:
                    return parity0

                return jax.lax.cond(i <= nn, do_row,
                                    lambda p0: p0, parity0)

            lax.fori_loop(1, N + 1, row_body, jnp.int32(0))

            @pl.when(sid == 0)
            def _():
                pltpu.sync_copy(x1_hbm, x1v)
                pltpu.sync_copy(y1_hbm, y1v)
                pltpu.sync_copy(x2_hbm, x2v)
                pltpu.sync_copy(y2_hbm, y2v)
                pltpu.sync_copy(scal_hbm, scalv)

                def cola_body(c, _):
                    ds = pl.ds(c * 16, 16)
                    gidx = lane + c * 16
                    pc = pcol[ds]
                    ok = jnp.logical_and(pc > 0, gidx < mm)
                    plsc.store_scatter(cola, [jnp.maximum(pc - 1, 0)], gidx,
                                       mask=ok)
                    return 0

                lax.fori_loop(0, NCHUNK, cola_body, 0)

                def sscat_body(c, _):
                    ds = pl.ds(c * 16, 16)
                    gidx = lane + c * 16
                    val = gidx < nn
                    cx = jnp.where(val, cola[ds], N)
                    plsc.store_scatter(sbuf, [jnp.minimum(cx, N - 1)], gidx,
                                       mask=val)
                    return 0

                lax.fori_loop(0, NCHUNK, sscat_body, 0)

                def order_body(c, _):
                    ds = pl.ds(c * 16, 16)
                    orderb[ds] = lane + c * 16
                    return 0

                lax.fori_loop(0, NCHUNK, order_body, 0)

                def rank_body(c, run):
                    ds = pl.ds(c * 16, 16)
                    sv = sbuf[ds]
                    occ = sv != BIG
                    occ_i = occ.astype(jnp.int32)
                    cs = plsc.cumsum(occ_i)
                    rank = run + cs - occ_i
                    plsc.store_scatter(orderb, [jnp.minimum(rank, N - 1)],
                                       sv, mask=occ)
                    return run + jnp.sum(occ_i)

                lax.fori_loop(0, NCHUNK, rank_body, jnp.int32(0))

                sv2 = scalv[...]
                tl = sv2[0]
                td0 = sv2[1]

                def fin_init(c, _):
                    ds = pl.ds(c * 16, 16)
                    outxv[ds] = x1v[ds]
                    outyv[ds] = y1v[ds]
                    return 0

                lax.fori_loop(0, NCHUNK, fin_init, 0)

                def fin_body(c, _):
                    ds = pl.ds(c * 16, 16)
                    gidx = lane + c * 16
                    valid = gidx < nn
                    cx = jnp.where(valid, cola[ds], N)
                    odr = orderb[ds]
                    cxo = plsc.load_gather(cola, [jnp.minimum(odr, N - 1)])
                    cxo = jnp.where(odr < nn, cxo, N)
                    r = jnp.where(t, cxo, gidx)
                    cc = jnp.where(t, odr, cx)
                    p2 = plsc.load_gather(idx2b, [jnp.clip(r, 0, N - 1)])
                    p1 = plsc.load_gather(idx1b, [jnp.clip(cc, 0, N - 1)])
                    p2c = jnp.clip(p2, 0, N - 1)
                    p1c = jnp.clip(p1, 0, N - 1)
                    mx2 = plsc.load_gather(x2v, [p2c])
                    my2 = plsc.load_gather(y2v, [p2c])
                    mx1 = plsc.load_gather(x1v, [p1c])
                    my1 = plsc.load_gather(y1v, [p1c])
                    ex = mx1 + (mx1 - mx2) / tl * (jnp.float32(0.0) - td0)
                    ey = my1 + (my1 - my2) / tl * (jnp.float32(0.0) - td0)
                    okm = jnp.logical_and(valid, p1 < N)
                    plsc.store_scatter(outxv, [p1c], ex, mask=okm)
                    plsc.store_scatter(outyv, [p1c], ey, mask=okm)
                    return 0

                lax.fori_loop(0, NCHUNK, fin_body, 0)

                pltpu.sync_copy(outxv, outx_hbm)
                pltpu.sync_copy(outyv, outy_hbm)

    return k(cost, costt, mask2, mask1, x1, y1, x2, y2, scal)


_TEMPLATE = [(1, -1, -1), (1, 1, -1), (-1, 1, -1), (-1, -1, -1),
             (1, -1, 1), (1, 1, 1), (-1, 1, 1), (-1, -1, 1)]


def _corners_body(cpt_ref, ox_ref, oy_ref, out_ref):
    x = ox_ref[0:1, :]
    y = oy_ref[0:1, :]
    z = cpt_ref[2:3, :]
    h = cpt_ref[3:4, :]
    w = cpt_ref[4:5, :]
    ll = cpt_ref[5:6, :]
    yaw = cpt_ref[6:7, :]
    cy = jnp.cos(yaw)
    sy = jnp.sin(yaw)
    for kk, (tx, ty, tz) in enumerate(_TEMPLATE):
        cxk = ll * (0.5 * tx)
        cyk = w * (0.5 * ty)
        czk = h * (0.5 * tz)
        rx = cxk * cy - cyk * sy + x
        ry = cxk * sy + cyk * cy + y
        rz = czk + z
        out_ref[3 * kk + 0:3 * kk + 1, :] = rx
        out_ref[3 * kk + 1:3 * kk + 2, :] = ry
        out_ref[3 * kk + 2:3 * kk + 3, :] = rz


def _corners_stage(center_t, ox, oy):
    return pl.pallas_call(
        _corners_body,
        out_shape=jax.ShapeDtypeStruct((24, N), jnp.float32),
    )(center_t, ox.reshape(1, N), oy.reshape(1, N))


def kernel(center_past1, center_past2, time_diff):
    cp1 = center_past1[:, :2]
    cp2 = center_past2[:, :2]
    cost, costt, m2, m1 = _cost_stage(cp1, cp2)
    tl = time_diff[0] - time_diff[1]
    tl = jnp.where(tl == 0, jnp.float32(1.0), tl)
    scal = jnp.zeros((16,), jnp.float32).at[0].set(tl).at[1].set(time_diff[0])
    outx, outy = _sc_matcher(
        cost, costt, m2.reshape(N), m1.reshape(N),
        cp1[:, 0], cp1[:, 1], cp2[:, 0], cp2[:, 1], scal)
    center_out = jnp.concatenate(
        [outx[:, None], outy[:, None], center_past1[:, 2:]], axis=1)
    out24 = _corners_stage(center_out.T, outx, outy)
    corners = out24.reshape(8, 3, N).transpose(2, 0, 1)
    return center_out, corners


# speedup vs baseline: 34.0758x; 1.0882x over previous
import functools

import jax
import jax.numpy as jnp
from jax import lax
from jax.experimental import pallas as pl
from jax.experimental.pallas import tpu as pltpu
from jax.experimental.pallas import tpu_sc as plsc

N = 512
THRE = 20.0
INF = 1e18
BIG = 1 << 30
NCHUNK = N // 16
NT = 16
CPT = N // NT
_STAGE = 4


def _cost_body(cp1t_ref, cp2_ref, cp2t_ref, cp1_ref, cost_ref, costt_ref,
               m2_ref, m1_ref):
    x1r = cp1t_ref[0:1, :]
    y1r = cp1t_ref[1:2, :]
    x2c = cp2_ref[:, 0:1]
    y2c = cp2_ref[:, 1:2]
    dx = x2c - x1r
    dy = y2c - y1r
    cost = jnp.sqrt(dx * dx + dy * dy)
    cost_ref[...] = cost
    x2r = cp2t_ref[0:1, :]
    y2r = cp2t_ref[1:2, :]
    x1c = cp1_ref[:, 0:1]
    y1c = cp1_ref[:, 1:2]
    dxt = x1c - x2r
    dyt = y1c - y2r
    costt = jnp.sqrt(dxt * dxt + dyt * dyt)
    costt_ref[...] = costt
    m2_ref[...] = (jnp.min(costt, axis=0, keepdims=True) <= THRE).astype(jnp.float32)
    m1_ref[...] = (jnp.min(cost, axis=0, keepdims=True) <= THRE).astype(jnp.float32)


def _cost_stage(cp1, cp2):
    return pl.pallas_call(
        _cost_body,
        out_shape=(
            jax.ShapeDtypeStruct((N, N), jnp.float32),
            jax.ShapeDtypeStruct((N, N), jnp.float32),
            jax.ShapeDtypeStruct((1, N), jnp.float32),
            jax.ShapeDtypeStruct((1, N), jnp.float32),
        ),
    )(cp1.T, cp2, cp2.T, cp1)


def _sload(ref, idx):
    return plsc.load_gather(ref, [jnp.full((16,), idx, jnp.int32)])[0]


def _sload2(ref, r, c):
    return plsc.load_gather(ref, [jnp.full((16,), r, jnp.int32),
                                  jnp.full((16,), c, jnp.int32)])[0]


def _sc_matcher(cost, costt, mask2, mask1, x1, y1, x2, y2, scal):
    mesh = plsc.VectorSubcoreMesh(core_axis_name="c", subcore_axis_name="s",
                                  num_cores=2, num_subcores=16)

    @functools.partial(
        pl.kernel,
        out_type=(
            jax.ShapeDtypeStruct((N,), jnp.float32),
            jax.ShapeDtypeStruct((N,), jnp.float32),
        ),
        mesh=mesh,
        compiler_params=pltpu.CompilerParams(needs_layout_passes=False),
        scratch_types=dict(
            p_sh=pltpu.VMEM_SHARED((2, NT, 128), jnp.float32),
            wam_sh=pltpu.VMEM_SHARED((NT, 128), jnp.int32),
            cw=pltpu.VMEM((N, CPT), jnp.float32),
            rbuf=pltpu.VMEM((64, N), jnp.float32),
            minvL=pltpu.VMEM((CPT,), jnp.float32),
            basefL=pltpu.VMEM((CPT,), jnp.float32),
            wayL=pltpu.VMEM((CPT,), jnp.int32),
            vcolL=pltpu.VMEM((CPT,), jnp.float32),
            amtL=pltpu.VMEM((CPT,), jnp.float32),
            wamF=pltpu.VMEM((NT, 128), jnp.int32),
            wam128=pltpu.VMEM((128,), jnp.int32),
            pcol=pltpu.VMEM((N,), jnp.int32),
            urow=pltpu.VMEM((N,), jnp.float32),
            colg=pltpu.VMEM((N,), jnp.int32),
            ridx=pltpu.VMEM((N,), jnp.int32),
            idx1b=pltpu.VMEM((N + 16,), jnp.int32),
            idx2b=pltpu.VMEM((N + 16,), jnp.int32),
            cola=pltpu.VMEM((N,), jnp.int32),
            sbuf=pltpu.VMEM((N,), jnp.int32),
            orderb=pltpu.VMEM((N,), jnp.int32),
            m1v=pltpu.VMEM((N,), jnp.float32),
            m2v=pltpu.VMEM((N,), jnp.float32),
            x1v=pltpu.VMEM((N,), jnp.float32),
            y1v=pltpu.VMEM((N,), jnp.float32),
            x2v=pltpu.VMEM((N,), jnp.float32),
            y2v=pltpu.VMEM((N,), jnp.float32),
            outxv=pltpu.VMEM((N,), jnp.float32),
            outyv=pltpu.VMEM((N,), jnp.float32),
            scalv=pltpu.VMEM((16,), jnp.float32),
            pbuf=pltpu.VMEM((128,), jnp.float32),
            xbuf=pltpu.VMEM((NT, 128), jnp.float32),
            sem=pltpu.SemaphoreType.DMA,
        ),
    )
    def k(cost_hbm, costt_hbm, m2_hbm, m1_hbm, x1_hbm, y1_hbm, x2_hbm,
          y2_hbm, scal_hbm, outx_hbm, outy_hbm, p_sh, wam_sh, cw, rbuf,
          minvL, basefL, wayL, vcolL, amtL, wamF, wam128, pcol, urow, colg,
          ridx, idx1b, idx2b, cola, sbuf, orderb, m1v, m2v, x1v, y1v, x2v,
          y2v, outxv, outyv, scalv, pbuf, xbuf, sem):
        cid = lax.axis_index("c")
        sid = lax.axis_index("s")
        lane = lax.iota(jnp.int32, 16)

        @pl.when(cid == 0)
        def _():
            pltpu.sync_copy(m2_hbm, m2v)
            pltpu.sync_copy(m1_hbm, m1v)

            def initpad_body(c, _):
                ds = pl.ds(c * 16, 16)
                idx1b[ds] = jnp.full((16,), N, jnp.int32)
                idx2b[ds] = jnp.full((16,), N, jnp.int32)
                return 0

            lax.fori_loop(0, NCHUNK + 1, initpad_body, 0)

            def comp_body(c, offs):
                o1, o2 = offs
                ds = pl.ds(c * 16, 16)
                gi = lane + c * 16
                mk1 = m1v[ds] > 0.5
                mk2 = m2v[ds] > 0.5
                plsc.store_compressed(idx1b.at[pl.ds(o1, 16)], gi, mask=mk1)
                plsc.store_compressed(idx2b.at[pl.ds(o2, 16)], gi, mask=mk2)
                o1 = o1 + jnp.sum(mk1.astype(jnp.int32))
                o2 = o2 + jnp.sum(mk2.astype(jnp.int32))
                return o1, o2

            n1, n2 = lax.fori_loop(0, NCHUNK, comp_body,
                                   (jnp.int32(0), jnp.int32(0)))
            t = n2 > n1
            nn = jnp.where(t, n1, n2)
            mm = jnp.where(t, n2, n1)

            def gid_body(c, _):
                ds = pl.ds(c * 16, 16)
                g1 = jnp.clip(idx1b[ds], 0, N - 1)
                g2 = jnp.clip(idx2b[ds], 0, N - 1)
                ridx[ds] = jnp.where(t, g1, g2)
                colg[ds] = jnp.where(t, g2, g1)
                pcol[ds] = jnp.zeros((16,), jnp.int32)
                urow[ds] = jnp.zeros((16,), jnp.float32)
                cola[ds] = jnp.zeros((16,), jnp.int32)
                sbuf[ds] = jnp.full((16,), BIG, jnp.int32)
                return 0

            lax.fori_loop(0, NCHUNK, gid_body, 0)

            colgA = colg[pl.ds(sid * CPT, 16)]
            colgB = colg[pl.ds(sid * CPT + 16, 16)]
            vcolL[pl.ds(0, 16)] = jnp.zeros((16,), jnp.float32)
            vcolL[pl.ds(16, 16)] = jnp.zeros((16,), jnp.float32)
            wayL[pl.ds(0, 16)] = jnp.zeros((16,), jnp.int32)
            wayL[pl.ds(16, 16)] = jnp.zeros((16,), jnp.int32)

            def wfinit_body(c, _):
                wamF[c, pl.ds(0, 16)] = jnp.zeros((16,), jnp.int32)
                wamF[c, pl.ds(16, 16)] = jnp.zeros((16,), jnp.int32)
                wamF[c, pl.ds(32, 16)] = jnp.zeros((16,), jnp.int32)
                wamF[c, pl.ds(48, 16)] = jnp.zeros((16,), jnp.int32)
                return 0

            lax.fori_loop(0, NT, wfinit_body, 0)
            for b in range(8):
                idx_slice = ridx.at[pl.ds(b * 64, 64)]

                @pl.when(t)
                def _():
                    pltpu.async_copy(costt_hbm.at[idx_slice], rbuf, sem).wait()

                @pl.when(jnp.logical_not(t))
                def _():
                    pltpu.async_copy(cost_hbm.at[idx_slice], rbuf, sem).wait()

                def stg_body(r8, _):
                    r8v = jnp.full((16,), r8, jnp.int32)
                    va = plsc.load_gather(rbuf, [r8v, colgA])
                    vb = plsc.load_gather(rbuf, [r8v, colgB])
                    rr = b * 64 + r8
                    cw[rr, pl.ds(0, 16)] = va
                    cw[rr, pl.ds(16, 16)] = vb
                    return 0

                lax.fori_loop(0, 64, stg_body, 0)

            gcolA = lane + sid * CPT
            gcolB = gcolA + 16

            def row_body(i, parity0):
                def do_row(parity0):
                    minvL[pl.ds(0, 16)] = jnp.full((16,), INF, jnp.float32)
                    minvL[pl.ds(16, 16)] = jnp.full((16,), INF, jnp.float32)
                    basefL[pl.ds(0, 16)] = jnp.where(
                        gcolA < mm, jnp.float32(INF), jnp.float32(0.0))
                    basefL[pl.ds(16, 16)] = jnp.where(
                        gcolB < mm, jnp.float32(INF), jnp.float32(0.0))

                    def wcond(carry):
                        cap = 520 if _STAGE < 4 else 1 << 20
                        return jnp.logical_and(jnp.logical_not(carry[4]),
                                               carry[5] < cap)

                    def wbody(carry):
                        j0, pending, dsum, parity, _, cnt = carry
                        own = (j0 - 1) // CPT

                        @pl.when(jnp.logical_and(j0 > 0, own == sid))
                        def _():
                            kloc = jnp.full((16,), (j0 - 1) % CPT, jnp.int32)
                            plsc.store_scatter(
                                basefL, [kloc],
                                jnp.full((16,), dsum, jnp.float32),
                                mask=lane < 1)

                        i0 = jnp.where(j0 > 0,
                                       _sload(pcol, jnp.maximum(j0 - 1, 0)),
                                       i)
                        u_i0 = _sload(urow, i0 - 1)
                        rr = i0 - 1

                        bsfA = basefL[pl.ds(0, 16)]
                        frmA = bsfA >= jnp.float32(INF)
                        curA = cw[rr, pl.ds(0, 16)] - u_i0 - vcolL[pl.ds(0, 16)]
                        mvA = minvL[pl.ds(0, 16)]
                        meA = jnp.where(frmA, mvA - pending, mvA)
                        upA = jnp.logical_and(frmA, curA < meA)
                        nmA = jnp.where(upA, curA, meA)
                        minvL[pl.ds(0, 16)] = nmA
                        wayL[pl.ds(0, 16)] = jnp.where(upA, j0,
                                                       wayL[pl.ds(0, 16)])
                        candA = jnp.where(frmA, nmA, jnp.float32(INF))

                        bsfB = basefL[pl.ds(16, 16)]
                        frmB = bsfB >= jnp.float32(INF)
                        curB = cw[rr, pl.ds(16, 16)] - u_i0 - vcolL[pl.ds(16, 16)]
                        mvB = minvL[pl.ds(16, 16)]
                        meB = jnp.where(frmB, mvB - pending, mvB)
                        upB = jnp.logical_and(frmB, curB < meB)
                        nmB = jnp.where(upB, curB, meB)
                        minvL[pl.ds(16, 16)] = nmB
                        wayL[pl.ds(16, 16)] = jnp.where(upB, j0,
                                                        wayL[pl.ds(16, 16)])
                        candB = jnp.where(frmB, nmB, jnp.float32(INF))

                        ltm = candB < candA
                        bestv = jnp.where(ltm, candB, candA)
                        besti = jnp.where(ltm, gcolB, gcolA)
                        lmin = jnp.min(bestv)
                        lidx = jnp.min(jnp.where(bestv == lmin, besti, BIG))

                        if _STAGE >= 4:
                            pv = jnp.where(lane < 1,
                                           jnp.full((16,), lmin, jnp.float32),
                                           plsc.bitcast(jnp.full((16,), lidx,
                                                                 jnp.int32),
                                                        jnp.float32))
                            pbuf[pl.ds(0, 16)] = pv
                            pltpu.sync_copy(pbuf, p_sh.at[parity, sid])
                            plsc.subcore_barrier()
                            pltpu.sync_copy(p_sh.at[parity], xbuf)
                            mins = plsc.load_gather(
                                xbuf, [lane, jnp.zeros((16,), jnp.int32)])
                            idxs = plsc.bitcast(
                                plsc.load_gather(
                                    xbuf, [lane, jnp.ones((16,), jnp.int32)]),
                                jnp.int32)
                            delta = jnp.min(mins)
                            jarg = jnp.min(jnp.where(mins == delta, idxs, BIG))
                        else:
                            if _STAGE == 3:
                                plsc.subcore_barrier()
                            delta = lmin
                            jarg = jnp.minimum(lidx, N - 1)
                        j1 = jarg + 1
                        done = _sload(pcol, jnp.minimum(jarg, N - 1)) == 0
                        if _STAGE == 3:
                            done = cnt >= 20
                        return (j1, delta, dsum + delta, 1 - parity, done,
                                cnt + 1)

                    j0f, _, dsum_f, parity_f, _, _ = lax.while_loop(
                        wcond, wbody,
                        (jnp.int32(0), jnp.float32(0.0), jnp.float32(0.0),
                         parity0, jnp.bool_(False), jnp.int32(0)))

                    plsc.addupdate_scatter(
                        urow, [jnp.full((16,), i - 1, jnp.int32)],
                        jnp.full((16,), dsum_f, jnp.float32),
                        mask=lane < 1)

                    usA = jnp.logical_and(basefL[pl.ds(0, 16)] < jnp.float32(INF),
                                          gcolA < mm)
                    aA = jnp.where(usA, dsum_f - basefL[pl.ds(0, 16)],
                                   jnp.float32(0.0))
                    vcolL[pl.ds(0, 16)] = vcolL[pl.ds(0, 16)] - aA
                    amtL[pl.ds(0, 16)] = aA
                    usB = jnp.logical_and(basefL[pl.ds(16, 16)] < jnp.float32(INF),
                                          gcolB < mm)
                    aB = jnp.where(usB, dsum_f - basefL[pl.ds(16, 16)],
                                   jnp.float32(0.0))
                    vcolL[pl.ds(16, 16)] = vcolL[pl.ds(16, 16)] - aB
                    amtL[pl.ds(16, 16)] = aB

                    wam128[pl.ds(0, 16)] = wayL[pl.ds(0, 16)]
                    wam128[pl.ds(16, 16)] = wayL[pl.ds(16, 16)]
                    wam128[pl.ds(32, 16)] = plsc.bitcast(amtL[pl.ds(0, 16)],
                                                         jnp.int32)
                    wam128[pl.ds(48, 16)] = plsc.bitcast(amtL[pl.ds(16, 16)],
                                                         jnp.int32)
                    if _STAGE >= 4:
                        pltpu.sync_copy(wam128, wam_sh.at[sid])
                        plsc.subcore_barrier()
                        pltpu.sync_copy(wam_sh, wamF)
                    else:
                        wamF[sid, pl.ds(0, 16)] = wam128[pl.ds(0, 16)]
                        wamF[sid, pl.ds(16, 16)] = wam128[pl.ds(16, 16)]
                        wamF[sid, pl.ds(32, 16)] = wam128[pl.ds(32, 16)]
                        wamF[sid, pl.ds(48, 16)] = wam128[pl.ds(48, 16)]

                    def uupd_body(c, _):
                        a16A = plsc.bitcast(wamF[c, pl.ds(32, 16)],
                                            jnp.float32)
                        a16B = plsc.bitcast(wamF[c, pl.ds(48, 16)],
                                            jnp.float32)
                        pcA = jnp.maximum(pcol[pl.ds(c * CPT, 16)] - 1, 0)
                        pcB = jnp.maximum(pcol[pl.ds(c * CPT + 16, 16)] - 1, 0)
                        plsc.addupdate_scatter(urow, [pcA], a16A,
                                               mask=a16A != 0)
                        plsc.addupdate_scatter(urow, [pcB], a16B,
                                               mask=a16B != 0)
                        return 0

                    lax.fori_loop(0, NT, uupd_body, 0)

                    def acond(carry):
                        return jnp.logical_and(carry[0] != 0,
                                               carry[1] < N + 2)

                    def abody(carry):
                        j0, hop = carry
                        jc = jnp.clip(j0 - 1, 0, N - 1)
                        jw = _sload2(wamF, jc // CPT, jc % CPT)
                        pv2 = jnp.where(
                            jw > 0, _sload(pcol, jnp.maximum(jw - 1, 0)), i)
                        plsc.store_scatter(
                            pcol, [jnp.full((16,), jc, jnp.int32)],
                            jnp.full((16,), pv2, jnp.int32),
                            mask=lane < 1)
                        return jw, hop + 1

                    lax.while_loop(acond, abody, (j0f, jnp.int32(0)))
                    if _STAGE >= 4:
                        plsc.subcore_barrier()
                    return parity_f

                if _STAGE < 2:
                    return parity0

                return jax.lax.cond(i <= nn, do_row,
                                    lambda p0: p0, parity0)

            lax.fori_loop(1, N + 1, row_body, jnp.int32(0))

            @pl.when(sid == 0)
            def _():
                pltpu.sync_copy(x1_hbm, x1v)
                pltpu.sync_copy(y1_hbm, y1v)
                pltpu.sync_copy(x2_hbm, x2v)
                pltpu.sync_copy(y2_hbm, y2v)
                pltpu.sync_copy(scal_hbm, scalv)

                def cola_body(c, _):
                    ds = pl.ds(c * 16, 16)
                    gidx = lane + c * 16
                    pc = pcol[ds]
                    ok = jnp.logical_and(pc > 0, gidx < mm)
                    plsc.store_scatter(cola, [jnp.maximum(pc - 1, 0)], gidx,
                                       mask=ok)
                    return 0

                lax.fori_loop(0, NCHUNK, cola_body, 0)

                def sscat_body(c, _):
                    ds = pl.ds(c * 16, 16)
                    gidx = lane + c * 16
                    val = gidx < nn
                    cx = jnp.where(val, cola[ds], N)
                    plsc.store_scatter(sbuf, [jnp.minimum(cx, N - 1)], gidx,
                                       mask=val)
                    return 0

                lax.fori_loop(0, NCHUNK, sscat_body, 0)

                def order_body(c, _):
                    ds = pl.ds(c * 16, 16)
                    orderb[ds] = lane + c * 16
                    return 0

                lax.fori_loop(0, NCHUNK, order_body, 0)

                def rank_body(c, run):
                    ds = pl.ds(c * 16, 16)
                    sv = sbuf[ds]
                    occ = sv != BIG
                    occ_i = occ.astype(jnp.int32)
                    cs = plsc.cumsum(occ_i)
                    rank = run + cs - occ_i
                    plsc.store_scatter(orderb, [jnp.minimum(rank, N - 1)],
                                       sv, mask=occ)
                    return run + jnp.sum(occ_i)

                lax.fori_loop(0, NCHUNK, rank_body, jnp.int32(0))

                sv2 = scalv[...]
                tl = sv2[0]
                td0 = sv2[1]

                def fin_init(c, _):
                    ds = pl.ds(c * 16, 16)
                    outxv[ds] = x1v[ds]
                    outyv[ds] = y1v[ds]
                    return 0

                lax.fori_loop(0, NCHUNK, fin_init, 0)

                def fin_body(c, _):
                    ds = pl.ds(c * 16, 16)
                    gidx = lane + c * 16
                    valid = gidx < nn
                    cx = jnp.where(valid, cola[ds], N)
                    odr = orderb[ds]
                    cxo = plsc.load_gather(cola, [jnp.minimum(odr, N - 1)])
                    cxo = jnp.where(odr < nn, cxo, N)
                    r = jnp.where(t, cxo, gidx)
                    cc = jnp.where(t, odr, cx)
                    p2 = plsc.load_gather(idx2b, [jnp.clip(r, 0, N - 1)])
                    p1 = plsc.load_gather(idx1b, [jnp.clip(cc, 0, N - 1)])
                    p2c = jnp.clip(p2, 0, N - 1)
                    p1c = jnp.clip(p1, 0, N - 1)
                    mx2 = plsc.load_gather(x2v, [p2c])
                    my2 = plsc.load_gather(y2v, [p2c])
                    mx1 = plsc.load_gather(x1v, [p1c])
                    my1 = plsc.load_gather(y1v, [p1c])
                    ex = mx1 + (mx1 - mx2) / tl * (jnp.float32(0.0) - td0)
                    ey = my1 + (my1 - my2) / tl * (jnp.float32(0.0) - td0)
                    okm = jnp.logical_and(valid, p1 < N)
                    plsc.store_scatter(outxv, [p1c], ex, mask=okm)
                    plsc.store_scatter(outyv, [p1c], ey, mask=okm)
                    return 0

                lax.fori_loop(0, NCHUNK, fin_body, 0)

                pltpu.sync_copy(outxv, outx_hbm)
                pltpu.sync_copy(outyv, outy_hbm)

    return k(cost, costt, mask2, mask1, x1, y1, x2, y2, scal)


_TEMPLATE = [(1, -1, -1), (1, 1, -1), (-1, 1, -1), (-1, -1, -1),
             (1, -1, 1), (1, 1, 1), (-1, 1, 1), (-1, -1, 1)]


def _corners_body(cpt_ref, ox_ref, oy_ref, out_ref):
    x = ox_ref[0:1, :]
    y = oy_ref[0:1, :]
    z = cpt_ref[2:3, :]
    h = cpt_ref[3:4, :]
    w = cpt_ref[4:5, :]
    ll = cpt_ref[5:6, :]
    yaw = cpt_ref[6:7, :]
    cy = jnp.cos(yaw)
    sy = jnp.sin(yaw)
    for kk, (tx, ty, tz) in enumerate(_TEMPLATE):
        cxk = ll * (0.5 * tx)
        cyk = w * (0.5 * ty)
        czk = h * (0.5 * tz)
        rx = cxk * cy - cyk * sy + x
        ry = cxk * sy + cyk * cy + y
        rz = czk + z
        out_ref[3 * kk + 0:3 * kk + 1, :] = rx
        out_ref[3 * kk + 1:3 * kk + 2, :] = ry
        out_ref[3 * kk + 2:3 * kk + 3, :] = rz


def _corners_stage(center_t, ox, oy):
    return pl.pallas_call(
        _corners_body,
        out_shape=jax.ShapeDtypeStruct((24, N), jnp.float32),
    )(center_t, ox.reshape(1, N), oy.reshape(1, N))


def kernel(center_past1, center_past2, time_diff):
    cp1 = center_past1[:, :2]
    cp2 = center_past2[:, :2]
    cost, costt, m2, m1 = _cost_stage(cp1, cp2)
    tl = time_diff[0] - time_diff[1]
    tl = jnp.where(tl == 0, jnp.float32(1.0), tl)
    scal = jnp.zeros((16,), jnp.float32).at[0].set(tl).at[1].set(time_diff[0])
    outx, outy = _sc_matcher(
        cost, costt, m2.reshape(N), m1.reshape(N),
        cp1[:, 0], cp1[:, 1], cp2[:, 0], cp2[:, 1], scal)
    center_out = jnp.concatenate(
        [outx[:, None], outy[:, None], center_past1[:, 2:]], axis=1)
    out24 = _corners_stage(center_out.T, outx, outy)
    corners = out24.reshape(8, 3, N).transpose(2, 0, 1)
    return center_out, corners


# speedup vs baseline: 34.1640x; 1.0026x over previous
import functools

import jax
import jax.numpy as jnp
from jax import lax
from jax.experimental import pallas as pl
from jax.experimental.pallas import tpu as pltpu
from jax.experimental.pallas import tpu_sc as plsc

N = 512
THRE = 20.0
INF = 1e18
BIG = 1 << 30
NCHUNK = N // 16
NT = 16
CPT = N // NT


def _cost_body(cp1t_ref, cp2_ref, cp2t_ref, cp1_ref, cost_ref, costt_ref,
               m2_ref, m1_ref):
    x1r = cp1t_ref[0:1, :]
    y1r = cp1t_ref[1:2, :]
    x2c = cp2_ref[:, 0:1]
    y2c = cp2_ref[:, 1:2]
    dx = x2c - x1r
    dy = y2c - y1r
    cost = jnp.sqrt(dx * dx + dy * dy)
    cost_ref[...] = cost
    x2r = cp2t_ref[0:1, :]
    y2r = cp2t_ref[1:2, :]
    x1c = cp1_ref[:, 0:1]
    y1c = cp1_ref[:, 1:2]
    dxt = x1c - x2r
    dyt = y1c - y2r
    costt = jnp.sqrt(dxt * dxt + dyt * dyt)
    costt_ref[...] = costt
    m2_ref[...] = (jnp.min(costt, axis=0, keepdims=True) <= THRE).astype(jnp.float32)
    m1_ref[...] = (jnp.min(cost, axis=0, keepdims=True) <= THRE).astype(jnp.float32)


def _cost_stage(cp1, cp2):
    return pl.pallas_call(
        _cost_body,
        out_shape=(
            jax.ShapeDtypeStruct((N, N), jnp.float32),
            jax.ShapeDtypeStruct((N, N), jnp.float32),
            jax.ShapeDtypeStruct((1, N), jnp.float32),
            jax.ShapeDtypeStruct((1, N), jnp.float32),
        ),
    )(cp1.T, cp2, cp2.T, cp1)


def _sload(ref, idx):
    return plsc.load_gather(ref, [jnp.full((16,), idx, jnp.int32)])[0]


def _sload2(ref, r, c):
    return plsc.load_gather(ref, [jnp.full((16,), r, jnp.int32),
                                  jnp.full((16,), c, jnp.int32)])[0]


def _sc_matcher(cost, costt, mask2, mask1, x1, y1, x2, y2, scal):
    mesh = plsc.VectorSubcoreMesh(core_axis_name="c", subcore_axis_name="s",
                                  num_cores=2, num_subcores=16)

    @functools.partial(
        pl.kernel,
        out_type=(
            jax.ShapeDtypeStruct((N,), jnp.float32),
            jax.ShapeDtypeStruct((N,), jnp.float32),
        ),
        mesh=mesh,
        compiler_params=pltpu.CompilerParams(needs_layout_passes=False),
        scratch_types=dict(
            p_sh=pltpu.VMEM_SHARED((2, NT, 128), jnp.float32),
            wam_sh=pltpu.VMEM_SHARED((NT, 128), jnp.int32),
            cw=pltpu.VMEM((N, CPT), jnp.float32),
            rbuf=pltpu.VMEM((64, N), jnp.float32),
            minvL=pltpu.VMEM((CPT,), jnp.float32),
            basefL=pltpu.VMEM((CPT,), jnp.float32),
            wayL=pltpu.VMEM((CPT,), jnp.int32),
            vcolL=pltpu.VMEM((CPT,), jnp.float32),
            amtL=pltpu.VMEM((CPT,), jnp.float32),
            wamF=pltpu.VMEM((NT, 128), jnp.int32),
            wam128=pltpu.VMEM((128,), jnp.int32),
            pcol=pltpu.VMEM((N,), jnp.int32),
            urow=pltpu.VMEM((N,), jnp.float32),
            colg=pltpu.VMEM((N,), jnp.int32),
            ridx=pltpu.VMEM((N,), jnp.int32),
            idx1b=pltpu.VMEM((N + 16,), jnp.int32),
            idx2b=pltpu.VMEM((N + 16,), jnp.int32),
            cola=pltpu.VMEM((N,), jnp.int32),
            sbuf=pltpu.VMEM((N,), jnp.int32),
            orderb=pltpu.VMEM((N,), jnp.int32),
            m1v=pltpu.VMEM((N,), jnp.float32),
            m2v=pltpu.VMEM((N,), jnp.float32),
            x1v=pltpu.VMEM((N,), jnp.float32),
            y1v=pltpu.VMEM((N,), jnp.float32),
            x2v=pltpu.VMEM((N,), jnp.float32),
            y2v=pltpu.VMEM((N,), jnp.float32),
            outxv=pltpu.VMEM((N,), jnp.float32),
            outyv=pltpu.VMEM((N,), jnp.float32),
            scalv=pltpu.VMEM((16,), jnp.float32),
            pbuf=pltpu.VMEM((128,), jnp.float32),
            xbuf=pltpu.VMEM((NT, 128), jnp.float32),
            sem=pltpu.SemaphoreType.DMA,
        ),
    )
    def k(cost_hbm, costt_hbm, m2_hbm, m1_hbm, x1_hbm, y1_hbm, x2_hbm,
          y2_hbm, scal_hbm, outx_hbm, outy_hbm, p_sh, wam_sh, cw, rbuf,
          minvL, basefL, wayL, vcolL, amtL, wamF, wam128, pcol, urow, colg,
          ridx, idx1b, idx2b, cola, sbuf, orderb, m1v, m2v, x1v, y1v, x2v,
          y2v, outxv, outyv, scalv, pbuf, xbuf, sem):
        cid = lax.axis_index("c")
        sid = lax.axis_index("s")
        lane = lax.iota(jnp.int32, 16)

        @pl.when(cid == 0)
        def _():
            pltpu.sync_copy(m2_hbm, m2v)
            pltpu.sync_copy(m1_hbm, m1v)

            def initpad_body(c, _):
                ds = pl.ds(c * 16, 16)
                idx1b[ds] = jnp.full((16,), N, jnp.int32)
                idx2b[ds] = jnp.full((16,), N, jnp.int32)
                return 0

            lax.fori_loop(0, NCHUNK + 1, initpad_body, 0)

            def comp_body(c, offs):
                o1, o2 = offs
                ds = pl.ds(c * 16, 16)
                gi = lane + c * 16
                mk1 = m1v[ds] > 0.5
                mk2 = m2v[ds] > 0.5
                plsc.store_compressed(idx1b.at[pl.ds(o1, 16)], gi, mask=mk1)
                plsc.store_compressed(idx2b.at[pl.ds(o2, 16)], gi, mask=mk2)
                o1 = o1 + jnp.sum(mk1.astype(jnp.int32))
                o2 = o2 + jnp.sum(mk2.astype(jnp.int32))
                return o1, o2

            n1, n2 = lax.fori_loop(0, NCHUNK, comp_body,
                                   (jnp.int32(0), jnp.int32(0)))
            t = n2 > n1
            nn = jnp.where(t, n1, n2)
            mm = jnp.where(t, n2, n1)

            def gid_body(c, _):
                ds = pl.ds(c * 16, 16)
                g1 = jnp.clip(idx1b[ds], 0, N - 1)
                g2 = jnp.clip(idx2b[ds], 0, N - 1)
                ridx[ds] = jnp.where(t, g1, g2)
                colg[ds] = jnp.where(t, g2, g1)
                pcol[ds] = jnp.zeros((16,), jnp.int32)
                urow[ds] = jnp.zeros((16,), jnp.float32)
                cola[ds] = jnp.zeros((16,), jnp.int32)
                sbuf[ds] = jnp.full((16,), BIG, jnp.int32)
                return 0

            lax.fori_loop(0, NCHUNK, gid_body, 0)

            colgA = colg[pl.ds(sid * CPT, 16)]
            colgB = colg[pl.ds(sid * CPT + 16, 16)]
            vcolL[pl.ds(0, 16)] = jnp.zeros((16,), jnp.float32)
            vcolL[pl.ds(16, 16)] = jnp.zeros((16,), jnp.float32)
            wayL[pl.ds(0, 16)] = jnp.zeros((16,), jnp.int32)
            wayL[pl.ds(16, 16)] = jnp.zeros((16,), jnp.int32)

            def wfinit_body(c, _):
                wamF[c, pl.ds(0, 16)] = jnp.zeros((16,), jnp.int32)
                wamF[c, pl.ds(16, 16)] = jnp.zeros((16,), jnp.int32)
                wamF[c, pl.ds(32, 16)] = jnp.zeros((16,), jnp.int32)
                wamF[c, pl.ds(48, 16)] = jnp.zeros((16,), jnp.int32)
                return 0

            lax.fori_loop(0, NT, wfinit_body, 0)
            for b in range(8):
                idx_slice = ridx.at[pl.ds(b * 64, 64)]

                @pl.when(t)
                def _():
                    pltpu.async_copy(costt_hbm.at[idx_slice], rbuf, sem).wait()

                @pl.when(jnp.logical_not(t))
                def _():
                    pltpu.async_copy(cost_hbm.at[idx_slice], rbuf, sem).wait()

                def stg_body(r8, _):
                    r8v = jnp.full((16,), r8, jnp.int32)
                    va = plsc.load_gather(rbuf, [r8v, colgA])
                    vb = plsc.load_gather(rbuf, [r8v, colgB])
                    rr = b * 64 + r8
                    cw[rr, pl.ds(0, 16)] = va
                    cw[rr, pl.ds(16, 16)] = vb
                    return 0

                lax.fori_loop(0, 64, stg_body, 0)

            gcolA = lane + sid * CPT
            gcolB = gcolA + 16

            def row_body(i, parity0):
                def do_row(parity0):
                    minvL[pl.ds(0, 16)] = jnp.full((16,), INF, jnp.float32)
                    minvL[pl.ds(16, 16)] = jnp.full((16,), INF, jnp.float32)
                    basefL[pl.ds(0, 16)] = jnp.where(
                        gcolA < mm, jnp.float32(INF), jnp.float32(0.0))
                    basefL[pl.ds(16, 16)] = jnp.where(
                        gcolB < mm, jnp.float32(INF), jnp.float32(0.0))

                    def wcond(carry):
                        return jnp.logical_and(jnp.logical_not(carry[4]),
                                               carry[5] < 1 << 20)

                    def wbody(carry):
                        j0, pending, dsum, parity, _, cnt = carry
                        own = (j0 - 1) // CPT

                        @pl.when(jnp.logical_and(j0 > 0, own == sid))
                        def _():
                            kloc = jnp.full((16,), (j0 - 1) % CPT, jnp.int32)
                            plsc.store_scatter(
                                basefL, [kloc],
                                jnp.full((16,), dsum, jnp.float32),
                                mask=lane < 1)

                        i0 = jnp.where(j0 > 0,
                                       _sload(pcol, jnp.maximum(j0 - 1, 0)),
                                       i)
                        u_i0 = _sload(urow, i0 - 1)
                        rr = i0 - 1

                        bsfA = basefL[pl.ds(0, 16)]
                        frmA = bsfA >= jnp.float32(INF)
                        curA = cw[rr, pl.ds(0, 16)] - u_i0 - vcolL[pl.ds(0, 16)]
                        mvA = minvL[pl.ds(0, 16)]
                        meA = jnp.where(frmA, mvA - pending, mvA)
                        upA = jnp.logical_and(frmA, curA < meA)
                        nmA = jnp.where(upA, curA, meA)
                        minvL[pl.ds(0, 16)] = nmA
                        wayL[pl.ds(0, 16)] = jnp.where(upA, j0,
                                                       wayL[pl.ds(0, 16)])
                        candA = jnp.where(frmA, nmA, jnp.float32(INF))

                        bsfB = basefL[pl.ds(16, 16)]
                        frmB = bsfB >= jnp.float32(INF)
                        curB = cw[rr, pl.ds(16, 16)] - u_i0 - vcolL[pl.ds(16, 16)]
                        mvB = minvL[pl.ds(16, 16)]
                        meB = jnp.where(frmB, mvB - pending, mvB)
                        upB = jnp.logical_and(frmB, curB < meB)
                        nmB = jnp.where(upB, curB, meB)
                        minvL[pl.ds(16, 16)] = nmB
                        wayL[pl.ds(16, 16)] = jnp.where(upB, j0,
                                                        wayL[pl.ds(16, 16)])
                        candB = jnp.where(frmB, nmB, jnp.float32(INF))

                        ltm = candB < candA
                        bestv = jnp.where(ltm, candB, candA)
                        besti = jnp.where(ltm, gcolB, gcolA)
                        lmin = jnp.min(bestv)
                        lidx = jnp.min(jnp.where(bestv == lmin, besti, BIG))

                        pv = jnp.where(lane < 1,
                                       jnp.full((16,), lmin, jnp.float32),
                                       plsc.bitcast(jnp.full((16,), lidx,
                                                             jnp.int32),
                                                    jnp.float32))
                        pbuf[pl.ds(0, 16)] = pv
                        pltpu.sync_copy(pbuf, p_sh.at[parity, sid])
                        plsc.subcore_barrier()
                        pltpu.sync_copy(p_sh.at[parity], xbuf)
                        mins = plsc.load_gather(
                            xbuf, [lane, jnp.zeros((16,), jnp.int32)])
                        idxs = plsc.bitcast(
                            plsc.load_gather(
                                xbuf, [lane, jnp.ones((16,), jnp.int32)]),
                            jnp.int32)
                        delta = jnp.min(mins)
                        jarg = jnp.min(jnp.where(mins == delta, idxs, BIG))
                        j1 = jarg + 1
                        done = _sload(pcol, jnp.minimum(jarg, N - 1)) == 0
                        return (j1, delta, dsum + delta, 1 - parity, done,
                                cnt + 1)

                    j0f, _, dsum_f, parity_f, _, _ = lax.while_loop(
                        wcond, wbody,
                        (jnp.int32(0), jnp.float32(0.0), jnp.float32(0.0),
                         parity0, jnp.bool_(False), jnp.int32(0)))

                    plsc.addupdate_scatter(
                        urow, [jnp.full((16,), i - 1, jnp.int32)],
                        jnp.full((16,), dsum_f, jnp.float32),
                        mask=lane < 1)

                    usA = jnp.logical_and(basefL[pl.ds(0, 16)] < jnp.float32(INF),
                                          gcolA < mm)
                    aA = jnp.where(usA, dsum_f - basefL[pl.ds(0, 16)],
                                   jnp.float32(0.0))
                    vcolL[pl.ds(0, 16)] = vcolL[pl.ds(0, 16)] - aA
                    amtL[pl.ds(0, 16)] = aA
                    usB = jnp.logical_and(basefL[pl.ds(16, 16)] < jnp.float32(INF),
                                          gcolB < mm)
                    aB = jnp.where(usB, dsum_f - basefL[pl.ds(16, 16)],
                                   jnp.float32(0.0))
                    vcolL[pl.ds(16, 16)] = vcolL[pl.ds(16, 16)] - aB
                    amtL[pl.ds(16, 16)] = aB

                    wam128[pl.ds(0, 16)] = wayL[pl.ds(0, 16)]
                    wam128[pl.ds(16, 16)] = wayL[pl.ds(16, 16)]
                    wam128[pl.ds(32, 16)] = plsc.bitcast(amtL[pl.ds(0, 16)],
                                                         jnp.int32)
                    wam128[pl.ds(48, 16)] = plsc.bitcast(amtL[pl.ds(16, 16)],
                                                         jnp.int32)
                    pltpu.sync_copy(wam128, wam_sh.at[sid])
                    plsc.subcore_barrier()
                    pltpu.sync_copy(wam_sh, wamF)

                    def uupd_body(c, _):
                        a16A = plsc.bitcast(wamF[c, pl.ds(32, 16)],
                                            jnp.float32)
                        a16B = plsc.bitcast(wamF[c, pl.ds(48, 16)],
                                            jnp.float32)
                        pcA = jnp.maximum(pcol[pl.ds(c * CPT, 16)] - 1, 0)
                        pcB = jnp.maximum(pcol[pl.ds(c * CPT + 16, 16)] - 1, 0)
                        plsc.addupdate_scatter(urow, [pcA], a16A,
                                               mask=a16A != 0)
                        plsc.addupdate_scatter(urow, [pcB], a16B,
                                               mask=a16B != 0)
                        return 0

                    lax.fori_loop(0, NT, uupd_body, 0)

                    def acond(carry):
                        return jnp.logical_and(carry[0] != 0,
                                               carry[1] < N + 2)

                    def abody(carry):
                        j0, hop = carry
                        jc = jnp.clip(j0 - 1, 0, N - 1)
                        jw = _sload2(wamF, jc // CPT, jc % CPT)
                        pv2 = jnp.where(
                            jw > 0, _sload(pcol, jnp.maximum(jw - 1, 0)), i)
                        plsc.store_scatter(
                            pcol, [jnp.full((16,), jc, jnp.int32)],
                            jnp.full((16,), pv2, jnp.int32),
                            mask=lane < 1)
                        return jw, hop + 1

                    lax.while_loop(acond, abody, (j0f, jnp.int32(0)))
                    plsc.subcore_barrier()
                    return parity_f

                return jax.lax.cond(i <= nn, do_row,
                                    lambda p0: p0, parity0)

            lax.fori_loop(1, N + 1, row_body, jnp.int32(0))

            @pl.when(sid == 0)
            def _():
                pltpu.sync_copy(x1_hbm, x1v)
                pltpu.sync_copy(y1_hbm, y1v)
                pltpu.sync_copy(x2_hbm, x2v)
                pltpu.sync_copy(y2_hbm, y2v)
                pltpu.sync_copy(scal_hbm, scalv)

                def cola_body(c, _):
                    ds = pl.ds(c * 16, 16)
                    gidx = lane + c * 16
                    pc = pcol[ds]
                    ok = jnp.logical_and(pc > 0, gidx < mm)
                    plsc.store_scatter(cola, [jnp.maximum(pc - 1, 0)], gidx,
                                       mask=ok)
                    return 0

                lax.fori_loop(0, NCHUNK, cola_body, 0)

                def sscat_body(c, _):
                    ds = pl.ds(c * 16, 16)
                    gidx = lane + c * 16
                    val = gidx < nn
                    cx = jnp.where(val, cola[ds], N)
                    plsc.store_scatter(sbuf, [jnp.minimum(cx, N - 1)], gidx,
                                       mask=val)
                    return 0

                lax.fori_loop(0, NCHUNK, sscat_body, 0)

                def order_body(c, _):
                    ds = pl.ds(c * 16, 16)
                    orderb[ds] = lane + c * 16
                    return 0

                lax.fori_loop(0, NCHUNK, order_body, 0)

                def rank_body(c, run):
                    ds = pl.ds(c * 16, 16)
                    sv = sbuf[ds]
                    occ = sv != BIG
                    occ_i = occ.astype(jnp.int32)
                    cs = plsc.cumsum(occ_i)
                    rank = run + cs - occ_i
                    plsc.store_scatter(orderb, [jnp.minimum(rank, N - 1)],
                                       sv, mask=occ)
                    return run + jnp.sum(occ_i)

                lax.fori_loop(0, NCHUNK, rank_body, jnp.int32(0))

                sv2 = scalv[...]
                tl = sv2[0]
                td0 = sv2[1]

                def fin_init(c, _):
                    ds = pl.ds(c * 16, 16)
                    outxv[ds] = x1v[ds]
                    outyv[ds] = y1v[ds]
                    return 0

                lax.fori_loop(0, NCHUNK, fin_init, 0)

                def fin_body(c, _):
                    ds = pl.ds(c * 16, 16)
                    gidx = lane + c * 16
                    valid = gidx < nn
                    cx = jnp.where(valid, cola[ds], N)
                    odr = orderb[ds]
                    cxo = plsc.load_gather(cola, [jnp.minimum(odr, N - 1)])
                    cxo = jnp.where(odr < nn, cxo, N)
                    r = jnp.where(t, cxo, gidx)
                    cc = jnp.where(t, odr, cx)
                    p2 = plsc.load_gather(idx2b, [jnp.clip(r, 0, N - 1)])
                    p1 = plsc.load_gather(idx1b, [jnp.clip(cc, 0, N - 1)])
                    p2c = jnp.clip(p2, 0, N - 1)
                    p1c = jnp.clip(p1, 0, N - 1)
                    mx2 = plsc.load_gather(x2v, [p2c])
                    my2 = plsc.load_gather(y2v, [p2c])
                    mx1 = plsc.load_gather(x1v, [p1c])
                    my1 = plsc.load_gather(y1v, [p1c])
                    ex = mx1 + (mx1 - mx2) / tl * (jnp.float32(0.0) - td0)
                    ey = my1 + (my1 - my2) / tl * (jnp.float32(0.0) - td0)
                    okm = jnp.logical_and(valid, p1 < N)
                    plsc.store_scatter(outxv, [p1c], ex, mask=okm)
                    plsc.store_scatter(outyv, [p1c], ey, mask=okm)
                    return 0

                lax.fori_loop(0, NCHUNK, fin_body, 0)

                pltpu.sync_copy(outxv, outx_hbm)
                pltpu.sync_copy(outyv, outy_hbm)

    return k(cost, costt, mask2, mask1, x1, y1, x2, y2, scal)


_TEMPLATE = [(1, -1, -1), (1, 1, -1), (-1, 1, -1), (-1, -1, -1),
             (1, -1, 1), (1, 1, 1), (-1, 1, 1), (-1, -1, 1)]


def _corners_body(cpt_ref, ox_ref, oy_ref, out_ref):
    x = ox_ref[0:1, :]
    y = oy_ref[0:1, :]
    z = cpt_ref[2:3, :]
    h = cpt_ref[3:4, :]
    w = cpt_ref[4:5, :]
    ll = cpt_ref[5:6, :]
    yaw = cpt_ref[6:7, :]
    cy = jnp.cos(yaw)
    sy = jnp.sin(yaw)
    for kk, (tx, ty, tz) in enumerate(_TEMPLATE):
        cxk = ll * (0.5 * tx)
        cyk = w * (0.5 * ty)
        czk = h * (0.5 * tz)
        rx = cxk * cy - cyk * sy + x
        ry = cxk * sy + cyk * cy + y
        rz = czk + z
        out_ref[3 * kk + 0:3 * kk + 1, :] = rx
        out_ref[3 * kk + 1:3 * kk + 2, :] = ry
        out_ref[3 * kk + 2:3 * kk + 3, :] = rz


def _corners_stage(center_t, ox, oy):
    return pl.pallas_call(
        _corners_body,
        out_shape=jax.ShapeDtypeStruct((24, N), jnp.float32),
    )(center_t, ox.reshape(1, N), oy.reshape(1, N))


def kernel(center_past1, center_past2, time_diff):
    cp1 = center_past1[:, :2]
    cp2 = center_past2[:, :2]
    cost, costt, m2, m1 = _cost_stage(cp1, cp2)
    tl = time_diff[0] - time_diff[1]
    tl = jnp.where(tl == 0, jnp.float32(1.0), tl)
    scal = jnp.zeros((16,), jnp.float32).at[0].set(tl).at[1].set(time_diff[0])
    outx, outy = _sc_matcher(
        cost, costt, m2.reshape(N), m1.reshape(N),
        cp1[:, 0], cp1[:, 1], cp2[:, 0], cp2[:, 1], scal)
    center_out = jnp.concatenate(
        [outx[:, None], outy[:, None], center_past1[:, 2:]], axis=1)
    out24 = _corners_stage(center_out.T, outx, outy)
    corners = out24.reshape(8, 3, N).transpose(2, 0, 1)
    return center_out, corners
